# baseline (speedup 1.0000x reference)
import functools

import numpy as np
import jax
import jax.numpy as jnp
from jax import lax
from jax.experimental import pallas as pl
from jax.experimental.pallas import tpu as pltpu
from jax.experimental.pallas import tpu_sc as plsc

N = 10000
C = 128
K = 32
Q = 4
NW = 32
NB = 4
NPW = 320
NPAD = NW * NPW
NBATCH = NPW // NB
EPAD = NPAD * K
KU = 4

_PERM = np.array([q * 32 + ch * 4 + m
                  for ch in range(8) for q in range(4) for m in range(4)],
                 dtype=np.int32)

_mesh = plsc.VectorSubcoreMesh(core_axis_name="c", subcore_axis_name="s")
_sc_params = pltpu.CompilerParams(needs_layout_passes=False)


def _dxdy_body(cx_hbm, cy_hbm, idx_hbm, dx_hbm, dy_hbm,
               cxv, cyv, idxv, dxv, dyv):
    wid = lax.axis_index("s") * 2 + lax.axis_index("c")
    pltpu.sync_copy(cx_hbm, cxv)
    pltpu.sync_copy(cy_hbm, cyv)
    ebase = wid * (NPW * K)
    pltpu.sync_copy(idx_hbm.at[pl.ds(ebase, NPW * K)], idxv)
    nbase = wid * NPW

    def node_step(nl, carry):
        n16 = jnp.full((16,), nbase + nl, jnp.int32)
        cx = plsc.load_gather(cxv, [n16])
        cy = plsc.load_gather(cyv, [n16])
        for h in range(K // 16):
            iv = idxv[pl.ds(nl * K + h * 16, 16)]
            gx = plsc.load_gather(cxv, [iv])
            gy = plsc.load_gather(cyv, [iv])
            dxv[pl.ds(nl * K + h * 16, 16)] = cx - gx
            dyv[pl.ds(nl * K + h * 16, 16)] = cy - gy
        return carry

    lax.fori_loop(0, NPW, node_step, 0)
    pltpu.sync_copy(dxv, dx_hbm.at[pl.ds(ebase, NPW * K)])
    pltpu.sync_copy(dyv, dy_hbm.at[pl.ds(ebase, NPW * K)])


_dxdy = functools.partial(
    pl.kernel,
    mesh=_mesh,
    out_type=[jax.ShapeDtypeStruct((EPAD,), jnp.float32),
              jax.ShapeDtypeStruct((EPAD,), jnp.float32)],
    scratch_types=[pltpu.VMEM((NPAD,), jnp.float32),
                   pltpu.VMEM((NPAD,), jnp.float32),
                   pltpu.VMEM((NPW * K,), jnp.int32),
                   pltpu.VMEM((NPW * K,), jnp.float32),
                   pltpu.VMEM((NPW * K,), jnp.float32)],
    compiler_params=_sc_params,
)(_dxdy_body)


def _wg_body(dx_ref, dy_ref, gw_ref, nf_ref, wcat_ref,
             mr_ref, mt_ref, pr_ref, pt_ref, ew_ref, g_ref):
    dx = dx_ref[...]
    dy = dy_ref[...]
    rho = jnp.sqrt(dx * dx + dy * dy)
    theta = jnp.arctan2(dx, dy)
    ws = []
    for q in range(Q):
        wr = jnp.exp(-0.5 * (rho - mr_ref[0, q]) ** 2
                     / (1e-14 + pr_ref[0, q] * pr_ref[0, q]))
        fa = jnp.abs(theta - mt_ref[0, q])
        sa = jnp.abs(2.0 * np.pi - fa)
        ang = jnp.minimum(fa, sa)
        wt = jnp.exp(-0.5 * ang * ang
                     / (1e-14 + pt_ref[0, q] * pt_ref[0, q]))
        w = wr * wt
        ws.append(jnp.where(jnp.isnan(w), 0.0, w))
    s = ws[0] + ws[1] + ws[2] + ws[3]
    gwb = gw_ref[...]
    ew_ref[...] = jnp.concatenate([gwb * (w / s) for w in ws], axis=1)
    g_ref[...] = jnp.dot(nf_ref[...], wcat_ref[...],
                         preferred_element_type=jnp.float32)


def _wg(dxp, dyp, gwp, nfp, wcat, mr, mt, pr, pt):
    grid = (NPAD // 128,)
    return pl.pallas_call(
        _wg_body,
        grid=grid,
        in_specs=[
            pl.BlockSpec((128, K), lambda i: (i, 0)),
            pl.BlockSpec((128, K), lambda i: (i, 0)),
            pl.BlockSpec((128, K), lambda i: (i, 0)),
            pl.BlockSpec((128, C), lambda i: (i, 0)),
            pl.BlockSpec((C, C), lambda i: (0, 0)),
            pl.BlockSpec(memory_space=pltpu.SMEM),
            pl.BlockSpec(memory_space=pltpu.SMEM),
            pl.BlockSpec(memory_space=pltpu.SMEM),
            pl.BlockSpec(memory_space=pltpu.SMEM),
        ],
        out_specs=[pl.BlockSpec((128, C), lambda i: (i, 0)),
                   pl.BlockSpec((128, C), lambda i: (i, 0))],
        out_shape=[jax.ShapeDtypeStruct((NPAD, C), jnp.float32),
                   jax.ShapeDtypeStruct((NPAD, C), jnp.float32)],
    )(dxp, dyp, gwp, nfp, wcat, mr, mt, pr, pt)


def _agg_body(g_hbm, idx3_hbm, ew_hbm, out_hbm,
              idxv, ewv, rows0, outv, sem0):
    wid = lax.axis_index("c") * 16 + lax.axis_index("s")
    pltpu.sync_copy(idx3_hbm.at[wid], idxv)
    pltpu.sync_copy(ew_hbm.at[pl.ds(wid * (NPW * C), NPW * C)], ewv)
    nbase = wid * NPW

    lanes = lax.iota(jnp.int32, 16)
    woff = (lanes // 4) * 32
    soff = woff + (lanes & 3)

    def compute(b, rows):
        for nb in range(NB):
            wb = (b * NB + nb) * C

            def kstep(kk, acc):
                accs = list(acc)
                for j in range(KU):
                    k = kk * KU + j
                    e = nb * K + k
                    wv = plsc.load_gather(
                        ewv, [jnp.full((16,), wb + k, jnp.int32) + woff])
                    for ch in range(C // 16):
                        accs[ch] = accs[ch] + wv * rows[e, pl.ds(ch * 16, 16)]
                return tuple(accs)

            acc0 = tuple(jnp.zeros((16,), jnp.float32) for _ in range(C // 16))
            acc = lax.fori_loop(0, K // KU, kstep, acc0)
            for ch in range(C // 16):
                plsc.store_scatter(
                    outv, [jnp.full((16,), nb * C + ch * 4, jnp.int32) + soff],
                    jnp.maximum(acc[ch], 0.0))
        pltpu.sync_copy(outv, out_hbm.at[pl.ds((nbase + b * NB) * C, NB * C)])

    def step(b, carry):
        cp = pltpu.make_async_copy(g_hbm.at[idxv.at[b]], rows0, sem0)
        cp.start()
        cp.wait()
        compute(b, rows0)
        return carry

    lax.fori_loop(0, NBATCH, step, 0)


_agg = functools.partial(
    pl.kernel,
    mesh=_mesh,
    out_type=jax.ShapeDtypeStruct((NPAD * C,), jnp.float32),
    scratch_types=[pltpu.VMEM((NBATCH, NB * K), jnp.int32),
                   pltpu.VMEM((NPW * C,), jnp.float32),
                   pltpu.VMEM((NB * K, C), jnp.float32),
                   pltpu.VMEM((NB * C,), jnp.float32),
                   pltpu.SemaphoreType.DMA],
    compiler_params=_sc_params,
)(_agg_body)


def kernel(node_feats, node_centre, neighbor_idx, graph_weights,
           mean_rho, mean_theta, precision_rho, precision_theta,
           W0, W1, W2, W3):
    nf = node_feats.reshape(N, C)
    cf = node_centre.reshape(N, 2)
    idx = neighbor_idx.astype(jnp.int32).reshape(N * K)
    gw = graph_weights.reshape(N, K)
    pad_n = NPAD - N
    nfp = jnp.pad(nf, ((0, pad_n), (0, 0)))
    cxp = jnp.pad(cf[:, 0], (0, pad_n))
    cyp = jnp.pad(cf[:, 1], (0, pad_n))
    idxp = jnp.pad(idx, (0, pad_n * K))
    gwp = jnp.pad(gw, ((0, pad_n), (0, 0)))
    wcat = jnp.concatenate([W0, W1, W2, W3], axis=1)[:, _PERM]

    dx1, dy1 = _dxdy(cxp, cyp, idxp)
    ew, g = _wg(dx1.reshape(NPAD, K), dy1.reshape(NPAD, K), gwp, nfp, wcat,
                mean_rho, mean_theta, precision_rho, precision_theta)
    out = _agg(g, idxp.reshape(NW, NBATCH, NB * K), ew.reshape(NPAD * C))
    return out.reshape(NPAD, C)[:N].reshape(1, N, C)


# speedup vs baseline: 1.2579x; 1.2579x over previous
import functools

import numpy as np
import jax
import jax.numpy as jnp
from jax import lax
from jax.experimental import pallas as pl
from jax.experimental.pallas import tpu as pltpu
from jax.experimental.pallas import tpu_sc as plsc

N = 10000
C = 128
K = 32
Q = 4
NW = 32
NB = 4
NPW = 320
NPAD = NW * NPW
NBATCH = NPW // NB
EPAD = NPAD * K
KU = 4

_PERM = np.array([q * 32 + ch * 4 + m
                  for ch in range(8) for q in range(4) for m in range(4)],
                 dtype=np.int32)

_mesh = plsc.VectorSubcoreMesh(core_axis_name="c", subcore_axis_name="s")
_sc_params = pltpu.CompilerParams(needs_layout_passes=False)


def _dxdy_body(cx_hbm, cy_hbm, idx_hbm, dx_hbm, dy_hbm,
               cxv, cyv, idxv, dxv, dyv):
    wid = lax.axis_index("s") * 2 + lax.axis_index("c")
    pltpu.sync_copy(cx_hbm, cxv)
    pltpu.sync_copy(cy_hbm, cyv)
    ebase = wid * (NPW * K)
    pltpu.sync_copy(idx_hbm.at[pl.ds(ebase, NPW * K)], idxv)
    nbase = wid * NPW

    def node_step(nl, carry):
        n16 = jnp.full((16,), nbase + nl, jnp.int32)
        cx = plsc.load_gather(cxv, [n16])
        cy = plsc.load_gather(cyv, [n16])
        for h in range(K // 16):
            iv = idxv[pl.ds(nl * K + h * 16, 16)]
            gx = plsc.load_gather(cxv, [iv])
            gy = plsc.load_gather(cyv, [iv])
            dxv[pl.ds(nl * K + h * 16, 16)] = cx - gx
            dyv[pl.ds(nl * K + h * 16, 16)] = cy - gy
        return carry

    lax.fori_loop(0, NPW, node_step, 0)
    pltpu.sync_copy(dxv, dx_hbm.at[pl.ds(ebase, NPW * K)])
    pltpu.sync_copy(dyv, dy_hbm.at[pl.ds(ebase, NPW * K)])


_dxdy = functools.partial(
    pl.kernel,
    mesh=_mesh,
    out_type=[jax.ShapeDtypeStruct((EPAD,), jnp.float32),
              jax.ShapeDtypeStruct((EPAD,), jnp.float32)],
    scratch_types=[pltpu.VMEM((NPAD,), jnp.float32),
                   pltpu.VMEM((NPAD,), jnp.float32),
                   pltpu.VMEM((NPW * K,), jnp.int32),
                   pltpu.VMEM((NPW * K,), jnp.float32),
                   pltpu.VMEM((NPW * K,), jnp.float32)],
    compiler_params=_sc_params,
)(_dxdy_body)


def _wg_body(dx_ref, dy_ref, gw_ref, nf_ref, wcat_ref,
             mr_ref, mt_ref, pr_ref, pt_ref, ew_ref, g_ref):
    dx = dx_ref[...]
    dy = dy_ref[...]
    rho = jnp.sqrt(dx * dx + dy * dy)
    theta = jnp.arctan2(dx, dy)
    ws = []
    for q in range(Q):
        wr = jnp.exp(-0.5 * (rho - mr_ref[0, q]) ** 2
                     / (1e-14 + pr_ref[0, q] * pr_ref[0, q]))
        fa = jnp.abs(theta - mt_ref[0, q])
        sa = jnp.abs(2.0 * np.pi - fa)
        ang = jnp.minimum(fa, sa)
        wt = jnp.exp(-0.5 * ang * ang
                     / (1e-14 + pt_ref[0, q] * pt_ref[0, q]))
        w = wr * wt
        ws.append(jnp.where(jnp.isnan(w), 0.0, w))
    s = ws[0] + ws[1] + ws[2] + ws[3]
    gwb = gw_ref[...]
    ew_ref[...] = jnp.concatenate([gwb * (w / s) for w in ws], axis=1)
    g_ref[...] = jnp.dot(nf_ref[...], wcat_ref[...],
                         preferred_element_type=jnp.float32)


def _wg(dxp, dyp, gwp, nfp, wcat, mr, mt, pr, pt):
    grid = (NPAD // 128,)
    return pl.pallas_call(
        _wg_body,
        grid=grid,
        in_specs=[
            pl.BlockSpec((128, K), lambda i: (i, 0)),
            pl.BlockSpec((128, K), lambda i: (i, 0)),
            pl.BlockSpec((128, K), lambda i: (i, 0)),
            pl.BlockSpec((128, C), lambda i: (i, 0)),
            pl.BlockSpec((C, C), lambda i: (0, 0)),
            pl.BlockSpec(memory_space=pltpu.SMEM),
            pl.BlockSpec(memory_space=pltpu.SMEM),
            pl.BlockSpec(memory_space=pltpu.SMEM),
            pl.BlockSpec(memory_space=pltpu.SMEM),
        ],
        out_specs=[pl.BlockSpec((128, C), lambda i: (i, 0)),
                   pl.BlockSpec((128, C), lambda i: (i, 0))],
        out_shape=[jax.ShapeDtypeStruct((NPAD, C), jnp.float32),
                   jax.ShapeDtypeStruct((NPAD, C), jnp.float32)],
    )(dxp, dyp, gwp, nfp, wcat, mr, mt, pr, pt)


def _agg_body(g_hbm, idx3_hbm, ew_hbm, out_hbm,
              idxv, ewv, rows0, rows1, outv, sem0, sem1):
    wid = lax.axis_index("c") * 16 + lax.axis_index("s")
    pltpu.sync_copy(idx3_hbm.at[wid], idxv)
    pltpu.sync_copy(ew_hbm.at[pl.ds(wid * (NPW * C), NPW * C)], ewv)
    nbase = wid * NPW

    lanes = lax.iota(jnp.int32, 16)
    woff = (lanes // 4) * 32
    soff = woff + (lanes & 3)

    def compute(b, rows):
        for nb in range(NB):
            wb = (b * NB + nb) * C

            def kstep(kk, acc):
                accs = list(acc)
                for j in range(KU):
                    k = kk * KU + j
                    e = nb * K + k
                    wv = plsc.load_gather(
                        ewv, [jnp.full((16,), wb + k, jnp.int32) + woff])
                    for ch in range(C // 16):
                        accs[ch] = accs[ch] + wv * rows[e, pl.ds(ch * 16, 16)]
                return tuple(accs)

            acc0 = tuple(jnp.zeros((16,), jnp.float32) for _ in range(C // 16))
            acc = lax.fori_loop(0, K // KU, kstep, acc0)
            for ch in range(C // 16):
                plsc.store_scatter(
                    outv, [jnp.full((16,), nb * C + ch * 4, jnp.int32) + soff],
                    jnp.maximum(acc[ch], 0.0))
        pltpu.sync_copy(outv, out_hbm.at[pl.ds((nbase + b * NB) * C, NB * C)])

    pltpu.make_async_copy(g_hbm.at[idxv.at[0]], rows0, sem0).start()

    def step(i, carry):
        b0 = 2 * i
        b1 = 2 * i + 1
        pltpu.make_async_copy(g_hbm.at[idxv.at[b1]], rows1, sem1).start()
        pltpu.make_async_copy(g_hbm.at[idxv.at[b0]], rows0, sem0).wait()
        compute(b0, rows0)

        @pl.when(i < NBATCH // 2 - 1)
        def _prefetch():
            pltpu.make_async_copy(
                g_hbm.at[idxv.at[b0 + 2]], rows0, sem0).start()

        pltpu.make_async_copy(g_hbm.at[idxv.at[b1]], rows1, sem1).wait()
        compute(b1, rows1)
        return carry

    lax.fori_loop(0, NBATCH // 2, step, 0)


_agg = functools.partial(
    pl.kernel,
    mesh=_mesh,
    out_type=jax.ShapeDtypeStruct((NPAD * C,), jnp.float32),
    scratch_types=[pltpu.VMEM((NBATCH, NB * K), jnp.int32),
                   pltpu.VMEM((NPW * C,), jnp.float32),
                   pltpu.VMEM((NB * K, C), jnp.float32),
                   pltpu.VMEM((NB * K, C), jnp.float32),
                   pltpu.VMEM((NB * C,), jnp.float32),
                   pltpu.SemaphoreType.DMA,
                   pltpu.SemaphoreType.DMA],
    compiler_params=_sc_params,
)(_agg_body)


def kernel(node_feats, node_centre, neighbor_idx, graph_weights,
           mean_rho, mean_theta, precision_rho, precision_theta,
           W0, W1, W2, W3):
    nf = node_feats.reshape(N, C)
    cf = node_centre.reshape(N, 2)
    idx = neighbor_idx.astype(jnp.int32).reshape(N * K)
    gw = graph_weights.reshape(N, K)
    pad_n = NPAD - N
    nfp = jnp.pad(nf, ((0, pad_n), (0, 0)))
    cxp = jnp.pad(cf[:, 0], (0, pad_n))
    cyp = jnp.pad(cf[:, 1], (0, pad_n))
    idxp = jnp.pad(idx, (0, pad_n * K))
    gwp = jnp.pad(gw, ((0, pad_n), (0, 0)))
    wcat = jnp.concatenate([W0, W1, W2, W3], axis=1)[:, _PERM]

    dx1, dy1 = _dxdy(cxp, cyp, idxp)
    ew, g = _wg(dx1.reshape(NPAD, K), dy1.reshape(NPAD, K), gwp, nfp, wcat,
                mean_rho, mean_theta, precision_rho, precision_theta)
    out = _agg(g, idxp.reshape(NW, NBATCH, NB * K), ew.reshape(NPAD * C))
    return out.reshape(NPAD, C)[:N].reshape(1, N, C)


# speedup vs baseline: 2.5385x; 2.0181x over previous
import functools

import numpy as np
import jax
import jax.numpy as jnp
from jax import lax
from jax.experimental import pallas as pl
from jax.experimental.pallas import tpu as pltpu
from jax.experimental.pallas import tpu_sc as plsc

N = 10000
C = 128
K = 32
Q = 4
NW = 32
NB = 4
NPW = 320
NPAD = NW * NPW
NBATCH = NPW // NB
EPAD = NPAD * K
KU = 4

_PERM = np.array([q * 32 + ch * 4 + m
                  for ch in range(8) for q in range(4) for m in range(4)],
                 dtype=np.int32)

_mesh = plsc.VectorSubcoreMesh(core_axis_name="c", subcore_axis_name="s")
_sc_params = pltpu.CompilerParams(needs_layout_passes=False)


def _dxdy_body(cx_hbm, cy_hbm, idx_hbm, dx_hbm, dy_hbm,
               cxv, cyv, idxv, dxv, dyv):
    wid = lax.axis_index("s") * 2 + lax.axis_index("c")
    pltpu.sync_copy(cx_hbm, cxv)
    pltpu.sync_copy(cy_hbm, cyv)
    ebase = wid * (NPW * K)
    pltpu.sync_copy(idx_hbm.at[pl.ds(ebase, NPW * K)], idxv)
    nbase = wid * NPW

    def node_step(nl, carry):
        n16 = jnp.full((16,), nbase + nl, jnp.int32)
        cx = plsc.load_gather(cxv, [n16])
        cy = plsc.load_gather(cyv, [n16])
        for h in range(K // 16):
            iv = idxv[pl.ds(nl * K + h * 16, 16)]
            gx = plsc.load_gather(cxv, [iv])
            gy = plsc.load_gather(cyv, [iv])
            dxv[pl.ds(nl * K + h * 16, 16)] = cx - gx
            dyv[pl.ds(nl * K + h * 16, 16)] = cy - gy
        return carry

    lax.fori_loop(0, NPW, node_step, 0)
    pltpu.sync_copy(dxv, dx_hbm.at[pl.ds(ebase, NPW * K)])
    pltpu.sync_copy(dyv, dy_hbm.at[pl.ds(ebase, NPW * K)])


_dxdy = functools.partial(
    pl.kernel,
    mesh=_mesh,
    out_type=[jax.ShapeDtypeStruct((EPAD,), jnp.float32),
              jax.ShapeDtypeStruct((EPAD,), jnp.float32)],
    scratch_types=[pltpu.VMEM((NPAD,), jnp.float32),
                   pltpu.VMEM((NPAD,), jnp.float32),
                   pltpu.VMEM((NPW * K,), jnp.int32),
                   pltpu.VMEM((NPW * K,), jnp.float32),
                   pltpu.VMEM((NPW * K,), jnp.float32)],
    compiler_params=_sc_params,
)(_dxdy_body)


def _wg_body(dx_ref, dy_ref, gw_ref, nf_ref, wcat_ref,
             mr_ref, mt_ref, pr_ref, pt_ref, ew_ref, g_ref):
    dx = dx_ref[...]
    dy = dy_ref[...]
    rho = jnp.sqrt(dx * dx + dy * dy)
    theta = jnp.arctan2(dx, dy)
    ws = []
    for q in range(Q):
        wr = jnp.exp(-0.5 * (rho - mr_ref[0, q]) ** 2
                     / (1e-14 + pr_ref[0, q] * pr_ref[0, q]))
        fa = jnp.abs(theta - mt_ref[0, q])
        sa = jnp.abs(2.0 * np.pi - fa)
        ang = jnp.minimum(fa, sa)
        wt = jnp.exp(-0.5 * ang * ang
                     / (1e-14 + pt_ref[0, q] * pt_ref[0, q]))
        w = wr * wt
        ws.append(jnp.where(jnp.isnan(w), 0.0, w))
    s = ws[0] + ws[1] + ws[2] + ws[3]
    gwb = gw_ref[...]
    ew_ref[...] = jnp.concatenate([gwb * (w / s) for w in ws], axis=1)
    g_ref[...] = jnp.dot(nf_ref[...], wcat_ref[...],
                         preferred_element_type=jnp.float32)


def _wg(dxp, dyp, gwp, nfp, wcat, mr, mt, pr, pt):
    grid = (NPAD // 128,)
    return pl.pallas_call(
        _wg_body,
        grid=grid,
        in_specs=[
            pl.BlockSpec((128, K), lambda i: (i, 0)),
            pl.BlockSpec((128, K), lambda i: (i, 0)),
            pl.BlockSpec((128, K), lambda i: (i, 0)),
            pl.BlockSpec((128, C), lambda i: (i, 0)),
            pl.BlockSpec((C, C), lambda i: (0, 0)),
            pl.BlockSpec(memory_space=pltpu.SMEM),
            pl.BlockSpec(memory_space=pltpu.SMEM),
            pl.BlockSpec(memory_space=pltpu.SMEM),
            pl.BlockSpec(memory_space=pltpu.SMEM),
        ],
        out_specs=[pl.BlockSpec((128, C), lambda i: (i, 0)),
                   pl.BlockSpec((128, C), lambda i: (i, 0))],
        out_shape=[jax.ShapeDtypeStruct((NPAD, C), jnp.float32),
                   jax.ShapeDtypeStruct((NPAD, C), jnp.float32)],
    )(dxp, dyp, gwp, nfp, wcat, mr, mt, pr, pt)


def _agg_body(g_hbm, idx3_hbm, ew_hbm, out_hbm,
              idxv, ewv, rows0, rows1, outv, sem0, sem1):
    wid = lax.axis_index("c") * 16 + lax.axis_index("s")
    pltpu.sync_copy(idx3_hbm.at[wid], idxv)
    pltpu.sync_copy(ew_hbm.at[pl.ds(wid * (NPW * C), NPW * C)], ewv)
    nbase = wid * NPW
    nbat = jnp.where(wid == NW - 1, (N - NPW * (NW - 1)) // NB, NBATCH)

    lanes = lax.iota(jnp.int32, 16)
    woff = (lanes // 4) * 32
    soff = woff + (lanes & 3)

    def compute(b, rows):
        for nb in range(NB):
            wb = (b * NB + nb) * C

            def kstep(kk, acc):
                accs = list(acc)
                for j in range(KU):
                    k = kk * KU + j
                    e = nb * K + k
                    wv = plsc.load_gather(
                        ewv, [jnp.full((16,), wb + k, jnp.int32) + woff])
                    for ch in range(C // 16):
                        accs[ch] = accs[ch] + wv * rows[e, pl.ds(ch * 16, 16)]
                return tuple(accs)

            acc0 = tuple(jnp.zeros((16,), jnp.float32) for _ in range(C // 16))
            acc = lax.fori_loop(0, K // KU, kstep, acc0)
            for ch in range(C // 16):
                plsc.store_scatter(
                    outv, [jnp.full((16,), nb * C + ch * 4, jnp.int32) + soff],
                    jnp.maximum(acc[ch], 0.0))
        pltpu.sync_copy(outv, out_hbm.at[pl.ds((nbase + b * NB) * C, NB * C)])

    pltpu.make_async_copy(g_hbm.at[idxv.at[0]], rows0, sem0).start()

    def step(i, carry):
        b0 = 2 * i
        b1 = 2 * i + 1
        pltpu.make_async_copy(g_hbm.at[idxv.at[b1]], rows1, sem1).start()
        pltpu.make_async_copy(g_hbm.at[idxv.at[b0]], rows0, sem0).wait()
        compute(b0, rows0)

        @pl.when(i < nbat // 2 - 1)
        def _prefetch():
            pltpu.make_async_copy(
                g_hbm.at[idxv.at[b0 + 2]], rows0, sem0).start()

        pltpu.make_async_copy(g_hbm.at[idxv.at[b1]], rows1, sem1).wait()
        compute(b1, rows1)
        return carry

    lax.fori_loop(0, nbat // 2, step, 0)


_agg = functools.partial(
    pl.kernel,
    mesh=_mesh,
    out_type=jax.ShapeDtypeStruct((N * C,), jnp.float32),
    scratch_types=[pltpu.VMEM((NBATCH, NB * K), jnp.int32),
                   pltpu.VMEM((NPW * C,), jnp.float32),
                   pltpu.VMEM((NB * K, C), jnp.float32),
                   pltpu.VMEM((NB * K, C), jnp.float32),
                   pltpu.VMEM((NB * C,), jnp.float32),
                   pltpu.SemaphoreType.DMA,
                   pltpu.SemaphoreType.DMA],
    compiler_params=_sc_params,
)(_agg_body)


def kernel(node_feats, node_centre, neighbor_idx, graph_weights,
           mean_rho, mean_theta, precision_rho, precision_theta,
           W0, W1, W2, W3):
    nf = node_feats.reshape(N, C)
    cf = node_centre.reshape(N, 2)
    idx = neighbor_idx.astype(jnp.int32).reshape(N * K)
    gw = graph_weights.reshape(N, K)
    pad_n = NPAD - N
    nfp = jnp.pad(nf, ((0, pad_n), (0, 0)))
    cxp = jnp.pad(cf[:, 0], (0, pad_n))
    cyp = jnp.pad(cf[:, 1], (0, pad_n))
    idxp = jnp.pad(idx, (0, pad_n * K))
    gwp = jnp.pad(gw, ((0, pad_n), (0, 0)))
    wcat = jnp.concatenate([W0, W1, W2, W3], axis=1)[:, _PERM]

    dx1, dy1 = _dxdy(cxp, cyp, idxp)
    ew, g = _wg(dx1.reshape(NPAD, K), dy1.reshape(NPAD, K), gwp, nfp, wcat,
                mean_rho, mean_theta, precision_rho, precision_theta)
    out = _agg(g, idxp.reshape(NW, NBATCH, NB * K), ew.reshape(NPAD * C))
    return out.reshape(1, N, C)
